# baseline (speedup 1.0000x reference)
import functools

import jax
import jax.numpy as jnp
from jax import lax
from jax.experimental import pallas as pl
from jax.experimental.pallas import tpu as pltpu, tpu_sc as plsc

NUM_FIXED_ROWS = 900000
NUM_TRAIN_ROWS = 100000
DIM = 64

NC, NS, L = 2, 16, 16
NW = NC * NS

B_TOTAL = 4096 * 200
PER_W = B_TOTAL // NW
CH = 256
NV = CH // L
N_CHUNKS = PER_W // CH
TMAX = PER_W + CH + L


def _sc_body(inp_hbm, fixed_hbm, train_hbm, out_hbm,
             idx_all, rows0, rows1, tpos1d, tpos_st, idxb_st,
             semg, semw0, semw1):
    wid = lax.axis_index("s") * NC + lax.axis_index("c")
    base = wid * PER_W
    lanes = lax.iota(jnp.int32, L)

    pltpu.sync_copy(inp_hbm.at[pl.ds(base, PER_W)], idx_all)

    def drain_write(rows_b, semw_b):
        pltpu.make_async_copy(rows_b, out_hbm.at[pl.ds(0, CH)],
                              semw_b).wait()

    pltpu.async_copy(rows0, out_hbm.at[pl.ds(base, CH)], semw0)
    pltpu.async_copy(rows1, out_hbm.at[pl.ds(base + CH, CH)], semw1)

    def step(j, n, rows_b, semw_b):
        drain_write(rows_b, semw_b)
        descs = []
        for k in range(NV):
            idx = idx_all[pl.ds(j * CH + k * L, L)]
            is_t = idx >= NUM_FIXED_ROWS
            idxf = jnp.minimum(idx, NUM_FIXED_ROWS - 1)
            descs.append(pltpu.async_copy(
                fixed_hbm.at[idxf], rows_b.at[pl.ds(k * L, L)], semg))
            key = jnp.where(is_t, lanes, lanes + L)
            gpos = lanes + (base + j * CH + k * L)
            _, pos_s = plsc.sort_key_val(key, gpos)
            tpos1d[pl.ds(n, L)] = pos_s
            n = n + jnp.sum(is_t.astype(jnp.int32))
        for d in descs:
            d.wait()
        pltpu.async_copy(rows_b, out_hbm.at[pl.ds(base + j * CH, CH)],
                         semw_b)
        return n

    def pair_body(i, n):
        n = step(2 * i, n, rows0, semw0)
        n = step(2 * i + 1, n, rows1, semw1)
        return n

    n = lax.fori_loop(0, N_CHUNKS // 2, pair_body, jnp.int32(0))
    drain_write(rows0, semw0)
    drain_write(rows1, semw1)

    first = tpos1d[pl.ds(0, L)]
    e0 = jnp.sum(jnp.where(lanes == 0, first, 0))
    evec = jnp.zeros((L,), jnp.int32) + e0

    def pad_body(i, _):
        tpos1d[pl.ds(n + i * L, L)] = evec
        return 0

    lax.fori_loop(0, CH // L, pad_body, 0)

    nch_b = (n + CH - 1) // CH

    def pass_b_step(j, _):
        for k in range(NV):
            tpos_st[pl.ds(k * L, L)] = tpos1d[pl.ds(j * CH + k * L, L)]
        pltpu.async_copy(inp_hbm.at[tpos_st], idxb_st, semg).wait()
        gd = []
        for k in range(NV):
            tix = idxb_st[pl.ds(k * L, L)] - NUM_FIXED_ROWS
            gd.append(pltpu.async_copy(
                train_hbm.at[tix], rows0.at[pl.ds(k * L, L)], semg))
        for d in gd:
            d.wait()
        pltpu.async_copy(rows0, out_hbm.at[tpos_st], semg).wait()
        return 0

    lax.fori_loop(0, nch_b, pass_b_step, 0)


@jax.jit
def _sc_lookup(inp_flat, fixed_weights, trainable_weight):
    mesh = plsc.VectorSubcoreMesh(
        core_axis_name="c", subcore_axis_name="s",
        num_cores=NC, num_subcores=NS)
    fn = pl.kernel(
        _sc_body,
        out_type=jax.ShapeDtypeStruct((B_TOTAL, DIM), jnp.float32),
        mesh=mesh,
        scratch_types=[
            pltpu.VMEM((PER_W,), jnp.int32),
            pltpu.VMEM((CH, DIM), jnp.float32),
            pltpu.VMEM((CH, DIM), jnp.float32),
            pltpu.VMEM((TMAX,), jnp.int32),
            pltpu.VMEM((CH,), jnp.int32),
            pltpu.VMEM((CH,), jnp.int32),
            pltpu.SemaphoreType.DMA,
            pltpu.SemaphoreType.DMA,
            pltpu.SemaphoreType.DMA,
        ],
        compiler_params=pltpu.CompilerParams(
            use_tc_tiling_on_sc=False, needs_layout_passes=False),
    )
    return fn(inp_flat, fixed_weights, trainable_weight)


def kernel(inp, fixed_weights, trainable_weight):
    inp_flat = inp.reshape(-1).astype(jnp.int32)
    out = _sc_lookup(inp_flat, fixed_weights, trainable_weight)
    return out.reshape(inp.shape + (DIM,))
